# baseline (speedup 1.0000x reference)
import functools

import jax
import jax.numpy as jnp
from jax import lax
from jax.experimental import pallas as pl
from jax.experimental.pallas import tpu as pltpu
from jax.experimental.pallas import tpu_sc as plsc

N = 10000
NPAD = 10240
E = 320000
D = 128

NC, NS = 2, 16
NW = NC * NS
CH = 40
NCH = 250
NB = 5
NI = 10
RPT = NPAD // NS

_mesh = plsc.VectorSubcoreMesh(core_axis_name="c", subcore_axis_name="s")


@functools.partial(
    pl.kernel,
    mesh=_mesh,
    out_type=jax.ShapeDtypeStruct((NC, NPAD), jnp.float32),
    scratch_types=[
        pltpu.VMEM((NCH, CH), jnp.int32),
        pltpu.VMEM((CH,), jnp.float32),
        pltpu.VMEM_SHARED((NPAD,), jnp.float32),
        pltpu.SemaphoreType.DMA,
    ],
)
def _sc_deg(dst_hbm, zeros1_hbm, out_hbm, dst_all, ones_v, acc, sem):
    cid = lax.axis_index("c")
    sid = lax.axis_index("s")
    wid = cid * NS + sid
    r0 = sid * RPT
    pltpu.sync_copy(zeros1_hbm.at[pl.ds(r0, RPT)], acc.at[pl.ds(r0, RPT)])
    pltpu.sync_copy(dst_hbm.at[wid], dst_all)
    for i in range(CH // 16):
        ones_v[pl.ds(i * 16, 16)] = jnp.full((16,), 1.0, jnp.float32)
    plsc.subcore_barrier()

    def fire(c, carry):
        pltpu.async_copy(ones_v, acc.at[dst_all.at[c]], sem, add=True)
        return carry

    lax.fori_loop(0, NCH, fire, 0)

    def drain(c, carry):
        pltpu.make_async_copy(ones_v, acc.at[dst_all.at[c]], sem).wait()
        return carry

    lax.fori_loop(0, NCH, drain, 0)
    plsc.subcore_barrier()
    pltpu.sync_copy(acc.at[pl.ds(r0, RPT)], out_hbm.at[cid, pl.ds(r0, RPT)])


@functools.partial(
    pl.kernel,
    mesh=_mesh,
    out_type=jax.ShapeDtypeStruct((NC, NPAD, D), jnp.float32),
    scratch_types=(
        [pltpu.VMEM((CH,), jnp.int32) for _ in range(2 * NI)]
        + [pltpu.VMEM((CH, D), jnp.float32) for _ in range(NB)]
        + [pltpu.VMEM_SHARED((NPAD, D), jnp.float32)]
        + [pltpu.SemaphoreType.DMA for _ in range(2 * NB + NI)]
    ),
)
def _sc_agg(hs_hbm, src_hbm, dst_hbm, zeros2_hbm, out_hbm, *refs):
    isrc = refs[0:NI]
    idst = refs[NI:2 * NI]
    rows = refs[2 * NI:2 * NI + NB]
    acc = refs[2 * NI + NB]
    gs = refs[2 * NI + NB + 1:2 * NI + 2 * NB + 1]
    ss = refs[2 * NI + 2 * NB + 1:2 * NI + 3 * NB + 1]
    isem = refs[2 * NI + 3 * NB + 1:]
    cid = lax.axis_index("c")
    sid = lax.axis_index("s")
    wid = cid * NS + sid
    rbase = sid * RPT
    pltpu.sync_copy(zeros2_hbm.at[pl.ds(rbase, RPT)],
                    acc.at[pl.ds(rbase, RPT)])
    plsc.subcore_barrier()

    def start_idx(c, k):
        pltpu.async_copy(src_hbm.at[wid, c], isrc[k], isem[k])
        pltpu.async_copy(dst_hbm.at[wid, c], idst[k], isem[k])

    def wait_idx(c, k):
        pltpu.make_async_copy(src_hbm.at[wid, c], isrc[k], isem[k]).wait()
        pltpu.make_async_copy(dst_hbm.at[wid, c], idst[k], isem[k]).wait()

    def start_gather(c, j, k):
        pltpu.async_copy(hs_hbm.at[isrc[k]], rows[j], gs[j])

    def wait_gather(c, j, k):
        pltpu.make_async_copy(hs_hbm.at[isrc[k]], rows[j], gs[j]).wait()

    def start_scatter(c, j, k):
        pltpu.async_copy(rows[j], acc.at[idst[k]], ss[j], add=True)

    def wait_scatter(c, j, k):
        pltpu.make_async_copy(rows[j], acc.at[idst[k]], ss[j]).wait()

    for c in range(8):
        start_idx(c, c)
    for c in range(3):
        wait_idx(c, c)
        start_gather(c, c, c)

    def body(t, carry):
        for u in range(NI):
            c = t * NI + u
            jr = u % NB
            wait_gather(c, jr, u % NI)
            start_scatter(c, jr, u % NI)

            @pl.when(c >= 2)
            def _():
                wait_scatter(c - 2, (u + 3) % NB, (u + 8) % NI)

            @pl.when(c + 3 < NCH)
            def _():
                wait_idx(c + 3, (u + 3) % NI)
                start_gather(c + 3, (u + 3) % NB, (u + 3) % NI)

            @pl.when(c + 8 < NCH)
            def _():
                start_idx(c + 8, (u + 8) % NI)
        return carry

    lax.fori_loop(0, NCH // NI, body, 0)
    wait_scatter(NCH - 2, (NCH - 2) % NB, (NCH - 2) % NI)
    wait_scatter(NCH - 1, (NCH - 1) % NB, (NCH - 1) % NI)
    plsc.subcore_barrier()
    pltpu.sync_copy(acc.at[pl.ds(rbase, RPT)],
                    out_hbm.at[cid, pl.ds(rbase, RPT)])


_R = 1000
_GRID = N // _R


def _tc1_body(degp0, degp1, x, w1, hs, dinv):
    d = degp0[0] + degp1[0] + 1.0
    di = lax.rsqrt(d)
    h = jnp.dot(x[...], w1[...], preferred_element_type=jnp.float32)
    hs[...] = h * di
    dinv[...] = di


def _tc1(degp, x, w1):
    return pl.pallas_call(
        _tc1_body,
        grid=(_GRID,),
        in_specs=[
            pl.BlockSpec((1, _R, 1), lambda i: (0, i, 0)),
            pl.BlockSpec((1, _R, 1), lambda i: (1, i, 0)),
            pl.BlockSpec((_R, D), lambda i: (i, 0)),
            pl.BlockSpec((D, D), lambda i: (0, 0)),
        ],
        out_specs=[
            pl.BlockSpec((_R, D), lambda i: (i, 0)),
            pl.BlockSpec((_R, 1), lambda i: (i, 0)),
        ],
        out_shape=[
            jax.ShapeDtypeStruct((N, D), jnp.float32),
            jax.ShapeDtypeStruct((N, 1), jnp.float32),
        ],
    )(degp, degp, x, w1)


def _tc2_body(e0, e1, hs, dinv, b1, w2, out):
    agg = e0[0] + e1[0] + hs[...]
    h1 = jnp.maximum(agg * dinv[...] + b1[...], 0.0)
    out[...] = jnp.dot(h1, w2[...],
                       preferred_element_type=jnp.float32) * dinv[...]


def _tc2(e, hs, dinv, b1, w2):
    return pl.pallas_call(
        _tc2_body,
        grid=(_GRID,),
        in_specs=[
            pl.BlockSpec((1, _R, D), lambda i: (0, i, 0)),
            pl.BlockSpec((1, _R, D), lambda i: (1, i, 0)),
            pl.BlockSpec((_R, D), lambda i: (i, 0)),
            pl.BlockSpec((_R, 1), lambda i: (i, 0)),
            pl.BlockSpec((D,), lambda i: (0,)),
            pl.BlockSpec((D, D), lambda i: (0, 0)),
        ],
        out_specs=pl.BlockSpec((_R, D), lambda i: (i, 0)),
        out_shape=jax.ShapeDtypeStruct((N, D), jnp.float32),
    )(e, e, hs, dinv, b1, w2)


def _tc3_body(e0, e1, hs, dinv, b2, out):
    out[...] = (e0[0] + e1[0] + hs[...]) * dinv[...] + b2[...]


def _tc3(e, hs, dinv, b2):
    return pl.pallas_call(
        _tc3_body,
        grid=(_GRID,),
        in_specs=[
            pl.BlockSpec((1, _R, D), lambda i: (0, i, 0)),
            pl.BlockSpec((1, _R, D), lambda i: (1, i, 0)),
            pl.BlockSpec((_R, D), lambda i: (i, 0)),
            pl.BlockSpec((_R, 1), lambda i: (i, 0)),
            pl.BlockSpec((D,), lambda i: (0,)),
        ],
        out_specs=pl.BlockSpec((_R, D), lambda i: (i, 0)),
        out_shape=jax.ShapeDtypeStruct((N, D), jnp.float32),
    )(e, e, hs, dinv, b2)


def kernel(x, edge_index, W1, b1, W2, b2):
    srcp = edge_index[0].reshape(NW, NCH, CH)
    dstp = edge_index[1].reshape(NW, NCH, CH)
    zeros1 = jnp.zeros((NPAD,), jnp.float32)
    zeros2 = jnp.zeros((NPAD, D), jnp.float32)

    degp = _sc_deg(dstp, zeros1).reshape(NC, NPAD, 1)
    hs1, dinv = _tc1(degp, x, W1)
    e1 = _sc_agg(hs1, srcp, dstp, zeros2)
    hs2 = _tc2(e1, hs1, dinv, b1, W2)
    e2 = _sc_agg(hs2, srcp, dstp, zeros2)
    return _tc3(e2, hs2, dinv, b2)


# speedup vs baseline: 1.0503x; 1.0503x over previous
import functools

import jax
import jax.numpy as jnp
from jax import lax
from jax.experimental import pallas as pl
from jax.experimental.pallas import tpu as pltpu
from jax.experimental.pallas import tpu_sc as plsc

N = 10000
NPAD = 10240
E = 320000
D = 128

NC, NS = 2, 16
NW = NC * NS
CH = 64
NCH = 160
NROW = E // CH
NB = 5
NI = 10
RPT = NPAD // NS

_mesh = plsc.VectorSubcoreMesh(core_axis_name="c", subcore_axis_name="s")


@functools.partial(
    pl.kernel,
    mesh=_mesh,
    out_type=jax.ShapeDtypeStruct((NC, NPAD), jnp.float32),
    scratch_types=[
        pltpu.VMEM((NCH, CH), jnp.int32),
        pltpu.VMEM((CH,), jnp.float32),
        pltpu.VMEM_SHARED((NPAD,), jnp.float32),
        pltpu.SemaphoreType.DMA,
    ],
)
def _sc_deg(dst_hbm, zeros1_hbm, out_hbm, dst_all, ones_v, acc, sem):
    cid = lax.axis_index("c")
    sid = lax.axis_index("s")
    wid = cid * NS + sid
    nch_w = jnp.where(wid == NW - 1, NCH // 4, NCH)
    r0 = sid * RPT
    pltpu.sync_copy(zeros1_hbm.at[pl.ds(r0, RPT)], acc.at[pl.ds(r0, RPT)])
    q = NCH // 4
    pltpu.sync_copy(dst_hbm.at[pl.ds(wid * NCH, q)], dst_all.at[pl.ds(0, q)])

    @pl.when(wid != NW - 1)
    def _():
        for i in range(1, 4):
            pltpu.sync_copy(dst_hbm.at[pl.ds(wid * NCH + i * q, q)],
                            dst_all.at[pl.ds(i * q, q)])

    for i in range(CH // 16):
        ones_v[pl.ds(i * 16, 16)] = jnp.full((16,), 1.0, jnp.float32)
    plsc.subcore_barrier()

    def fire(c, carry):
        pltpu.async_copy(ones_v, acc.at[dst_all.at[c]], sem, add=True)
        return carry

    lax.fori_loop(0, nch_w, fire, 0)

    def drain(c, carry):
        pltpu.make_async_copy(ones_v, acc.at[dst_all.at[c]], sem).wait()
        return carry

    lax.fori_loop(0, nch_w, drain, 0)
    plsc.subcore_barrier()
    pltpu.sync_copy(acc.at[pl.ds(r0, RPT)], out_hbm.at[cid, pl.ds(r0, RPT)])


@functools.partial(
    pl.kernel,
    mesh=_mesh,
    out_type=jax.ShapeDtypeStruct((NC, NPAD, D), jnp.float32),
    scratch_types=(
        [pltpu.VMEM((CH,), jnp.int32) for _ in range(2 * NI)]
        + [pltpu.VMEM((CH, D), jnp.float32) for _ in range(NB)]
        + [pltpu.VMEM_SHARED((NPAD, D), jnp.float32)]
        + [pltpu.SemaphoreType.DMA for _ in range(2 * NB + NI)]
    ),
)
def _sc_agg(hs_hbm, src_hbm, dst_hbm, zeros2_hbm, out_hbm, *refs):
    isrc = refs[0:NI]
    idst = refs[NI:2 * NI]
    rows = refs[2 * NI:2 * NI + NB]
    acc = refs[2 * NI + NB]
    gs = refs[2 * NI + NB + 1:2 * NI + 2 * NB + 1]
    ss = refs[2 * NI + 2 * NB + 1:2 * NI + 3 * NB + 1]
    isem = refs[2 * NI + 3 * NB + 1:]
    cid = lax.axis_index("c")
    sid = lax.axis_index("s")
    wid = cid * NS + sid
    nch_w = jnp.where(wid == NW - 1, NCH // 4, NCH)
    rbase = sid * RPT
    pltpu.sync_copy(zeros2_hbm.at[pl.ds(rbase, RPT)],
                    acc.at[pl.ds(rbase, RPT)])
    plsc.subcore_barrier()

    def start_idx(c, k):
        pltpu.async_copy(src_hbm.at[wid * NCH + c], isrc[k], isem[k])
        pltpu.async_copy(dst_hbm.at[wid * NCH + c], idst[k], isem[k])

    def wait_idx(c, k):
        pltpu.make_async_copy(src_hbm.at[wid * NCH + c], isrc[k],
                              isem[k]).wait()
        pltpu.make_async_copy(dst_hbm.at[wid * NCH + c], idst[k],
                              isem[k]).wait()

    def start_gather(c, j, k):
        pltpu.async_copy(hs_hbm.at[isrc[k]], rows[j], gs[j])

    def wait_gather(c, j, k):
        pltpu.make_async_copy(hs_hbm.at[isrc[k]], rows[j], gs[j]).wait()

    def start_scatter(c, j, k):
        pltpu.async_copy(rows[j], acc.at[idst[k]], ss[j], add=True)

    def wait_scatter(c, j, k):
        pltpu.make_async_copy(rows[j], acc.at[idst[k]], ss[j]).wait()

    for c in range(8):
        start_idx(c, c)
    for c in range(3):
        wait_idx(c, c)
        start_gather(c, c, c)

    def body(t, carry):
        for u in range(NI):
            c = t * NI + u
            jr = u % NB
            wait_gather(c, jr, u % NI)
            start_scatter(c, jr, u % NI)

            @pl.when(c >= 2)
            def _():
                wait_scatter(c - 2, (u + 3) % NB, (u + 8) % NI)

            @pl.when(c + 3 < nch_w)
            def _():
                wait_idx(c + 3, (u + 3) % NI)
                start_gather(c + 3, (u + 3) % NB, (u + 3) % NI)

            @pl.when(c + 8 < nch_w)
            def _():
                start_idx(c + 8, (u + 8) % NI)
        return carry

    lax.fori_loop(0, nch_w // NI, body, 0)
    wait_scatter(nch_w - 2, (NCH - 2) % NB, (NCH - 2) % NI)
    wait_scatter(nch_w - 1, (NCH - 1) % NB, (NCH - 1) % NI)
    plsc.subcore_barrier()
    pltpu.sync_copy(acc.at[pl.ds(rbase, RPT)],
                    out_hbm.at[cid, pl.ds(rbase, RPT)])


_R = 1000
_GRID = N // _R


def _tc1_body(degp0, degp1, x, w1, hs, dinv):
    d = degp0[0] + degp1[0] + 1.0
    di = lax.rsqrt(d)
    h = jnp.dot(x[...], w1[...], preferred_element_type=jnp.float32)
    hs[...] = h * di
    dinv[...] = di


def _tc1(degp, x, w1):
    return pl.pallas_call(
        _tc1_body,
        grid=(_GRID,),
        in_specs=[
            pl.BlockSpec((1, _R, 1), lambda i: (0, i, 0)),
            pl.BlockSpec((1, _R, 1), lambda i: (1, i, 0)),
            pl.BlockSpec((_R, D), lambda i: (i, 0)),
            pl.BlockSpec((D, D), lambda i: (0, 0)),
        ],
        out_specs=[
            pl.BlockSpec((_R, D), lambda i: (i, 0)),
            pl.BlockSpec((_R, 1), lambda i: (i, 0)),
        ],
        out_shape=[
            jax.ShapeDtypeStruct((N, D), jnp.float32),
            jax.ShapeDtypeStruct((N, 1), jnp.float32),
        ],
    )(degp, degp, x, w1)


def _tc2_body(e0, e1, hs, dinv, b1, w2, out):
    agg = e0[0] + e1[0] + hs[...]
    h1 = jnp.maximum(agg * dinv[...] + b1[...], 0.0)
    out[...] = jnp.dot(h1, w2[...],
                       preferred_element_type=jnp.float32) * dinv[...]


def _tc2(e, hs, dinv, b1, w2):
    return pl.pallas_call(
        _tc2_body,
        grid=(_GRID,),
        in_specs=[
            pl.BlockSpec((1, _R, D), lambda i: (0, i, 0)),
            pl.BlockSpec((1, _R, D), lambda i: (1, i, 0)),
            pl.BlockSpec((_R, D), lambda i: (i, 0)),
            pl.BlockSpec((_R, 1), lambda i: (i, 0)),
            pl.BlockSpec((D,), lambda i: (0,)),
            pl.BlockSpec((D, D), lambda i: (0, 0)),
        ],
        out_specs=pl.BlockSpec((_R, D), lambda i: (i, 0)),
        out_shape=jax.ShapeDtypeStruct((N, D), jnp.float32),
    )(e, e, hs, dinv, b1, w2)


def _tc3_body(e0, e1, hs, dinv, b2, out):
    out[...] = (e0[0] + e1[0] + hs[...]) * dinv[...] + b2[...]


def _tc3(e, hs, dinv, b2):
    return pl.pallas_call(
        _tc3_body,
        grid=(_GRID,),
        in_specs=[
            pl.BlockSpec((1, _R, D), lambda i: (0, i, 0)),
            pl.BlockSpec((1, _R, D), lambda i: (1, i, 0)),
            pl.BlockSpec((_R, D), lambda i: (i, 0)),
            pl.BlockSpec((_R, 1), lambda i: (i, 0)),
            pl.BlockSpec((D,), lambda i: (0,)),
        ],
        out_specs=pl.BlockSpec((_R, D), lambda i: (i, 0)),
        out_shape=jax.ShapeDtypeStruct((N, D), jnp.float32),
    )(e, e, hs, dinv, b2)


def kernel(x, edge_index, W1, b1, W2, b2):
    srcp = edge_index[0].reshape(NROW, CH)
    dstp = edge_index[1].reshape(NROW, CH)
    zeros1 = jnp.zeros((NPAD,), jnp.float32)
    zeros2 = jnp.zeros((NPAD, D), jnp.float32)

    degp = _sc_deg(dstp, zeros1).reshape(NC, NPAD, 1)
    hs1, dinv = _tc1(degp, x, W1)
    e1 = _sc_agg(hs1, srcp, dstp, zeros2)
    hs2 = _tc2(e1, hs1, dinv, b1, W2)
    e2 = _sc_agg(hs2, srcp, dstp, zeros2)
    return _tc3(e2, hs2, dinv, b2)


# speedup vs baseline: 1.0723x; 1.0210x over previous
import functools

import jax
import jax.numpy as jnp
from jax import lax
from jax.experimental import pallas as pl
from jax.experimental.pallas import tpu as pltpu
from jax.experimental.pallas import tpu_sc as plsc

N = 10000
NPAD = 10240
E = 320000
D = 128

NC, NS = 2, 16
NW = NC * NS
CH = 64
NCH = 160
NROW = E // CH
NB = 5
NI = 10
RPT = NPAD // NS

_mesh = plsc.VectorSubcoreMesh(core_axis_name="c", subcore_axis_name="s")


@functools.partial(
    pl.kernel,
    mesh=_mesh,
    out_type=jax.ShapeDtypeStruct((NC, NPAD), jnp.float32),
    scratch_types=[
        pltpu.VMEM((NCH, CH), jnp.int32),
        pltpu.VMEM((CH,), jnp.float32),
        pltpu.VMEM_SHARED((NPAD,), jnp.float32),
        pltpu.SemaphoreType.DMA,
    ],
)
def _sc_deg(dst_hbm, zeros1_hbm, out_hbm, dst_all, ones_v, acc, sem):
    cid = lax.axis_index("c")
    sid = lax.axis_index("s")
    wid = cid * NS + sid
    nch_w = jnp.where(wid == NW - 1, NCH // 4, NCH)
    r0 = sid * RPT
    pltpu.sync_copy(zeros1_hbm.at[pl.ds(r0, RPT)], acc.at[pl.ds(r0, RPT)])
    q = NCH // 4
    pltpu.sync_copy(dst_hbm.at[pl.ds(wid * NCH, q)], dst_all.at[pl.ds(0, q)])

    @pl.when(wid != NW - 1)
    def _():
        for i in range(1, 4):
            pltpu.sync_copy(dst_hbm.at[pl.ds(wid * NCH + i * q, q)],
                            dst_all.at[pl.ds(i * q, q)])

    for i in range(CH // 16):
        ones_v[pl.ds(i * 16, 16)] = jnp.full((16,), 1.0, jnp.float32)
    plsc.subcore_barrier()

    def fire(c, carry):
        pltpu.async_copy(ones_v, acc.at[dst_all.at[c]], sem, add=True)
        return carry

    lax.fori_loop(0, nch_w, fire, 0)

    def drain(c, carry):
        pltpu.make_async_copy(ones_v, acc.at[dst_all.at[c]], sem).wait()
        return carry

    lax.fori_loop(0, nch_w, drain, 0)
    plsc.subcore_barrier()
    pltpu.sync_copy(acc.at[pl.ds(r0, RPT)], out_hbm.at[cid, pl.ds(r0, RPT)])


@functools.partial(
    pl.kernel,
    mesh=_mesh,
    out_type=jax.ShapeDtypeStruct((NC, NPAD, D), jnp.float32),
    scratch_types=(
        [pltpu.VMEM((CH,), jnp.int32) for _ in range(2 * NI)]
        + [pltpu.VMEM((CH, D), jnp.float32) for _ in range(NB)]
        + [pltpu.VMEM_SHARED((NPAD, D), jnp.float32)]
        + [pltpu.SemaphoreType.DMA for _ in range(2 * NB + NI)]
    ),
)
def _sc_agg(hs_hbm, src_hbm, dst_hbm, zeros2_hbm, out_hbm, *refs):
    isrc = refs[0:NI]
    idst = refs[NI:2 * NI]
    rows = refs[2 * NI:2 * NI + NB]
    acc = refs[2 * NI + NB]
    gs = refs[2 * NI + NB + 1:2 * NI + 2 * NB + 1]
    ss = refs[2 * NI + 2 * NB + 1:2 * NI + 3 * NB + 1]
    isem = refs[2 * NI + 3 * NB + 1:]
    cid = lax.axis_index("c")
    sid = lax.axis_index("s")
    wid = cid * NS + sid
    nch_w = jnp.where(wid == NW - 1, NCH // 4, NCH)
    rbase = sid * RPT
    pltpu.sync_copy(zeros2_hbm.at[pl.ds(rbase, RPT)],
                    acc.at[pl.ds(rbase, RPT)])
    plsc.subcore_barrier()

    def start_idx(c, k):
        pltpu.async_copy(src_hbm.at[wid * NCH + c], isrc[k], isem[k])
        pltpu.async_copy(dst_hbm.at[wid * NCH + c], idst[k], isem[k])

    def wait_idx(c, k):
        pltpu.make_async_copy(src_hbm.at[wid * NCH + c], isrc[k],
                              isem[k]).wait()
        pltpu.make_async_copy(dst_hbm.at[wid * NCH + c], idst[k],
                              isem[k]).wait()

    def start_gather(c, j, k):
        pltpu.async_copy(hs_hbm.at[isrc[k]], rows[j], gs[j])

    def wait_gather(c, j, k):
        pltpu.make_async_copy(hs_hbm.at[isrc[k]], rows[j], gs[j]).wait()

    def start_scatter(c, j, k):
        pltpu.async_copy(rows[j], acc.at[idst[k]], ss[j], add=True)

    def wait_scatter(c, j, k):
        pltpu.make_async_copy(rows[j], acc.at[idst[k]], ss[j]).wait()

    for c in range(8):
        start_idx(c, c)
    for c in range(3):
        wait_idx(c, c)
        start_gather(c, c, c)

    def body(t, carry):
        for u in range(NI):
            c = t * NI + u
            jr = u % NB
            wait_gather(c, jr, u % NI)
            start_scatter(c, jr, u % NI)

            @pl.when(c >= 2)
            def _():
                wait_scatter(c - 2, (u + 3) % NB, (u + 8) % NI)

            @pl.when(c + 3 < nch_w)
            def _():
                wait_idx(c + 3, (u + 3) % NI)
                start_gather(c + 3, (u + 3) % NB, (u + 3) % NI)

            @pl.when(c + 8 < nch_w)
            def _():
                start_idx(c + 8, (u + 8) % NI)
        return carry

    lax.fori_loop(0, nch_w // NI, body, 0)
    wait_scatter(nch_w - 2, (NCH - 2) % NB, (NCH - 2) % NI)
    wait_scatter(nch_w - 1, (NCH - 1) % NB, (NCH - 1) % NI)
    plsc.subcore_barrier()
    pltpu.sync_copy(acc.at[pl.ds(rbase, RPT)],
                    out_hbm.at[cid, pl.ds(rbase, RPT)])


_R = 2000
_GRID = N // _R


def _tc1_body(degp0, degp1, x, w1, hs, dinv):
    d = degp0[0] + degp1[0] + 1.0
    di = lax.rsqrt(d)
    h = jnp.dot(x[...], w1[...], preferred_element_type=jnp.float32)
    hs[...] = h * di
    dinv[...] = di


def _tc1(degp, x, w1):
    return pl.pallas_call(
        _tc1_body,
        grid=(_GRID,),
        in_specs=[
            pl.BlockSpec((1, _R, 1), lambda i: (0, i, 0)),
            pl.BlockSpec((1, _R, 1), lambda i: (1, i, 0)),
            pl.BlockSpec((_R, D), lambda i: (i, 0)),
            pl.BlockSpec((D, D), lambda i: (0, 0)),
        ],
        out_specs=[
            pl.BlockSpec((_R, D), lambda i: (i, 0)),
            pl.BlockSpec((_R, 1), lambda i: (i, 0)),
        ],
        out_shape=[
            jax.ShapeDtypeStruct((N, D), jnp.float32),
            jax.ShapeDtypeStruct((N, 1), jnp.float32),
        ],
    )(degp, degp, x, w1)


def _tc2_body(e0, e1, hs, dinv, b1, w2, out):
    agg = e0[0] + e1[0] + hs[...]
    h1 = jnp.maximum(agg * dinv[...] + b1[...], 0.0)
    out[...] = jnp.dot(h1, w2[...],
                       preferred_element_type=jnp.float32) * dinv[...]


def _tc2(e, hs, dinv, b1, w2):
    return pl.pallas_call(
        _tc2_body,
        grid=(_GRID,),
        in_specs=[
            pl.BlockSpec((1, _R, D), lambda i: (0, i, 0)),
            pl.BlockSpec((1, _R, D), lambda i: (1, i, 0)),
            pl.BlockSpec((_R, D), lambda i: (i, 0)),
            pl.BlockSpec((_R, 1), lambda i: (i, 0)),
            pl.BlockSpec((D,), lambda i: (0,)),
            pl.BlockSpec((D, D), lambda i: (0, 0)),
        ],
        out_specs=pl.BlockSpec((_R, D), lambda i: (i, 0)),
        out_shape=jax.ShapeDtypeStruct((N, D), jnp.float32),
    )(e, e, hs, dinv, b1, w2)


def _tc3_body(e0, e1, hs, dinv, b2, out):
    out[...] = (e0[0] + e1[0] + hs[...]) * dinv[...] + b2[...]


def _tc3(e, hs, dinv, b2):
    return pl.pallas_call(
        _tc3_body,
        grid=(_GRID,),
        in_specs=[
            pl.BlockSpec((1, _R, D), lambda i: (0, i, 0)),
            pl.BlockSpec((1, _R, D), lambda i: (1, i, 0)),
            pl.BlockSpec((_R, D), lambda i: (i, 0)),
            pl.BlockSpec((_R, 1), lambda i: (i, 0)),
            pl.BlockSpec((D,), lambda i: (0,)),
        ],
        out_specs=pl.BlockSpec((_R, D), lambda i: (i, 0)),
        out_shape=jax.ShapeDtypeStruct((N, D), jnp.float32),
    )(e, e, hs, dinv, b2)


def kernel(x, edge_index, W1, b1, W2, b2):
    srcp = edge_index[0].reshape(NROW, CH)
    dstp = edge_index[1].reshape(NROW, CH)
    zeros1 = jnp.zeros((NPAD,), jnp.float32)
    zeros2 = jnp.zeros((NPAD, D), jnp.float32)

    degp = _sc_deg(dstp, zeros1).reshape(NC, NPAD, 1)
    hs1, dinv = _tc1(degp, x, W1)
    e1 = _sc_agg(hs1, srcp, dstp, zeros2)
    hs2 = _tc2(e1, hs1, dinv, b1, W2)
    e2 = _sc_agg(hs2, srcp, dstp, zeros2)
    return _tc3(e2, hs2, dinv, b2)


# speedup vs baseline: 1.1052x; 1.0306x over previous
import functools

import jax
import jax.numpy as jnp
from jax import lax
from jax.experimental import pallas as pl
from jax.experimental.pallas import tpu as pltpu
from jax.experimental.pallas import tpu_sc as plsc

N = 10000
NPAD = 10240
E = 320000
D = 128

NC, NS = 2, 16
NW = NC * NS
CH = 64
NCH = 160
NROW = E // CH
NB = 5
NI = 10
RPT = NPAD // NS

_mesh = plsc.VectorSubcoreMesh(core_axis_name="c", subcore_axis_name="s")


@functools.partial(
    pl.kernel,
    mesh=_mesh,
    out_type=jax.ShapeDtypeStruct((NC, NPAD), jnp.float32),
    scratch_types=[
        pltpu.VMEM((NCH, CH), jnp.int32),
        pltpu.VMEM((CH,), jnp.float32),
        pltpu.VMEM_SHARED((NPAD,), jnp.float32),
        pltpu.SemaphoreType.DMA,
    ],
)
def _sc_deg(dst_hbm, zeros1_hbm, out_hbm, dst_all, ones_v, acc, sem):
    cid = lax.axis_index("c")
    sid = lax.axis_index("s")
    wid = cid * NS + sid
    nch_w = jnp.where(wid == NW - 1, NCH // 4, NCH)
    r0 = sid * RPT
    pltpu.sync_copy(zeros1_hbm.at[pl.ds(r0, RPT)], acc.at[pl.ds(r0, RPT)])
    q = NCH // 4
    pltpu.sync_copy(dst_hbm.at[pl.ds(wid * NCH, q)], dst_all.at[pl.ds(0, q)])

    @pl.when(wid != NW - 1)
    def _():
        for i in range(1, 4):
            pltpu.sync_copy(dst_hbm.at[pl.ds(wid * NCH + i * q, q)],
                            dst_all.at[pl.ds(i * q, q)])

    for i in range(CH // 16):
        ones_v[pl.ds(i * 16, 16)] = jnp.full((16,), 1.0, jnp.float32)
    plsc.subcore_barrier()

    def fire(c, carry):
        pltpu.async_copy(ones_v, acc.at[dst_all.at[c]], sem, add=True)
        return carry

    lax.fori_loop(0, nch_w, fire, 0)

    def drain(c, carry):
        pltpu.make_async_copy(ones_v, acc.at[dst_all.at[c]], sem).wait()
        return carry

    lax.fori_loop(0, nch_w, drain, 0)
    plsc.subcore_barrier()
    pltpu.sync_copy(acc.at[pl.ds(r0, RPT)], out_hbm.at[cid, pl.ds(r0, RPT)])


@functools.partial(
    pl.kernel,
    mesh=_mesh,
    out_type=jax.ShapeDtypeStruct((NC, NPAD, D), jnp.float32),
    scratch_types=(
        [pltpu.VMEM((CH,), jnp.int32) for _ in range(2 * NI)]
        + [pltpu.VMEM((CH, D), jnp.float32) for _ in range(NB)]
        + [pltpu.VMEM_SHARED((NPAD, D), jnp.float32)]
        + [pltpu.SemaphoreType.DMA for _ in range(2 * NB + NI)]
    ),
)
def _sc_agg(hs_hbm, src_hbm, dst_hbm, out_hbm, *refs):
    isrc = refs[0:NI]
    idst = refs[NI:2 * NI]
    rows = refs[2 * NI:2 * NI + NB]
    acc = refs[2 * NI + NB]
    gs = refs[2 * NI + NB + 1:2 * NI + 2 * NB + 1]
    ss = refs[2 * NI + 2 * NB + 1:2 * NI + 3 * NB + 1]
    isem = refs[2 * NI + 3 * NB + 1:]
    cid = lax.axis_index("c")
    sid = lax.axis_index("s")
    wid = cid * NS + sid
    nch_w = jnp.where(wid == NW - 1, NCH // 4, NCH)
    rbase = sid * RPT

    def zrow(r, carry):
        for i in range(D // 16):
            rows[0][r, pl.ds(i * 16, 16)] = jnp.zeros((16,), jnp.float32)
        return carry

    lax.fori_loop(0, CH, zrow, 0)

    def zcopy(q, carry):
        pltpu.sync_copy(rows[0], acc.at[pl.ds(rbase + q * CH, CH)])
        return carry

    lax.fori_loop(0, RPT // CH, zcopy, 0)
    plsc.subcore_barrier()

    def start_idx(c, k):
        pltpu.async_copy(src_hbm.at[wid * NCH + c], isrc[k], isem[k])
        pltpu.async_copy(dst_hbm.at[wid * NCH + c], idst[k], isem[k])

    def wait_idx(c, k):
        pltpu.make_async_copy(src_hbm.at[wid * NCH + c], isrc[k],
                              isem[k]).wait()
        pltpu.make_async_copy(dst_hbm.at[wid * NCH + c], idst[k],
                              isem[k]).wait()

    def start_gather(c, j, k):
        pltpu.async_copy(hs_hbm.at[isrc[k]], rows[j], gs[j])

    def wait_gather(c, j, k):
        pltpu.make_async_copy(hs_hbm.at[isrc[k]], rows[j], gs[j]).wait()

    def start_scatter(c, j, k):
        pltpu.async_copy(rows[j], acc.at[idst[k]], ss[j], add=True)

    def wait_scatter(c, j, k):
        pltpu.make_async_copy(rows[j], acc.at[idst[k]], ss[j]).wait()

    for c in range(8):
        start_idx(c, c)
    for c in range(3):
        wait_idx(c, c)
        start_gather(c, c, c)

    def body(t, carry):
        for u in range(NI):
            c = t * NI + u
            jr = u % NB
            wait_gather(c, jr, u % NI)
            start_scatter(c, jr, u % NI)

            @pl.when(c >= 2)
            def _():
                wait_scatter(c - 2, (u + 3) % NB, (u + 8) % NI)

            @pl.when(c + 3 < nch_w)
            def _():
                wait_idx(c + 3, (u + 3) % NI)
                start_gather(c + 3, (u + 3) % NB, (u + 3) % NI)

            @pl.when(c + 8 < nch_w)
            def _():
                start_idx(c + 8, (u + 8) % NI)
        return carry

    lax.fori_loop(0, nch_w // NI, body, 0)
    wait_scatter(nch_w - 2, (NCH - 2) % NB, (NCH - 2) % NI)
    wait_scatter(nch_w - 1, (NCH - 1) % NB, (NCH - 1) % NI)
    plsc.subcore_barrier()
    pltpu.sync_copy(acc.at[pl.ds(rbase, RPT)],
                    out_hbm.at[cid, pl.ds(rbase, RPT)])


_R = 2000
_GRID = N // _R


def _tc1_body(degp0, degp1, x, w1, hs, dinv):
    d = degp0[0] + degp1[0] + 1.0
    di = lax.rsqrt(d)
    h = jnp.dot(x[...], w1[...], preferred_element_type=jnp.float32)
    hs[...] = h * di
    dinv[...] = di


def _tc1(degp, x, w1):
    return pl.pallas_call(
        _tc1_body,
        grid=(_GRID,),
        in_specs=[
            pl.BlockSpec((1, _R, 1), lambda i: (0, i, 0)),
            pl.BlockSpec((1, _R, 1), lambda i: (1, i, 0)),
            pl.BlockSpec((_R, D), lambda i: (i, 0)),
            pl.BlockSpec((D, D), lambda i: (0, 0)),
        ],
        out_specs=[
            pl.BlockSpec((_R, D), lambda i: (i, 0)),
            pl.BlockSpec((_R, 1), lambda i: (i, 0)),
        ],
        out_shape=[
            jax.ShapeDtypeStruct((N, D), jnp.float32),
            jax.ShapeDtypeStruct((N, 1), jnp.float32),
        ],
    )(degp, degp, x, w1)


def _tc2_body(e0, e1, hs, dinv, b1, w2, out):
    agg = e0[0] + e1[0] + hs[...]
    h1 = jnp.maximum(agg * dinv[...] + b1[...], 0.0)
    out[...] = jnp.dot(h1, w2[...],
                       preferred_element_type=jnp.float32) * dinv[...]


def _tc2(e, hs, dinv, b1, w2):
    return pl.pallas_call(
        _tc2_body,
        grid=(_GRID,),
        in_specs=[
            pl.BlockSpec((1, _R, D), lambda i: (0, i, 0)),
            pl.BlockSpec((1, _R, D), lambda i: (1, i, 0)),
            pl.BlockSpec((_R, D), lambda i: (i, 0)),
            pl.BlockSpec((_R, 1), lambda i: (i, 0)),
            pl.BlockSpec((D,), lambda i: (0,)),
            pl.BlockSpec((D, D), lambda i: (0, 0)),
        ],
        out_specs=pl.BlockSpec((_R, D), lambda i: (i, 0)),
        out_shape=jax.ShapeDtypeStruct((N, D), jnp.float32),
    )(e, e, hs, dinv, b1, w2)


def _tc3_body(e0, e1, hs, dinv, b2, out):
    out[...] = (e0[0] + e1[0] + hs[...]) * dinv[...] + b2[...]


def _tc3(e, hs, dinv, b2):
    return pl.pallas_call(
        _tc3_body,
        grid=(_GRID,),
        in_specs=[
            pl.BlockSpec((1, _R, D), lambda i: (0, i, 0)),
            pl.BlockSpec((1, _R, D), lambda i: (1, i, 0)),
            pl.BlockSpec((_R, D), lambda i: (i, 0)),
            pl.BlockSpec((_R, 1), lambda i: (i, 0)),
            pl.BlockSpec((D,), lambda i: (0,)),
        ],
        out_specs=pl.BlockSpec((_R, D), lambda i: (i, 0)),
        out_shape=jax.ShapeDtypeStruct((N, D), jnp.float32),
    )(e, e, hs, dinv, b2)


def kernel(x, edge_index, W1, b1, W2, b2):
    srcp = edge_index[0].reshape(NROW, CH)
    dstp = edge_index[1].reshape(NROW, CH)
    zeros1 = jnp.zeros((NPAD,), jnp.float32)

    degp = _sc_deg(dstp, zeros1).reshape(NC, NPAD, 1)
    hs1, dinv = _tc1(degp, x, W1)
    e1 = _sc_agg(hs1, srcp, dstp)
    hs2 = _tc2(e1, hs1, dinv, b1, W2)
    e2 = _sc_agg(hs2, srcp, dstp)
    return _tc3(e2, hs2, dinv, b2)
